# Initial kernel scaffold; baseline (speedup 1.0000x reference)
import jax
import jax.numpy as jnp
from jax.experimental import pallas as pl


def kernel(x, embeddings):
    raise NotImplementedError("write your pallas kernel here")


# speedup vs baseline: 1.1031x; 1.1031x over previous
import functools

import jax
import jax.numpy as jnp
from jax import lax
from jax.experimental import pallas as pl
from jax.experimental.pallas import tpu as pltpu
from jax.experimental.pallas import tpu_sc as plsc


@functools.lru_cache(maxsize=None)
def _make_gather(V, D, B, CH):
    info = plsc.get_sparse_core_info()
    NC, NS = info.num_cores, info.num_subcores
    NW = NC * NS
    assert B % NW == 0
    b_per_w = B // NW
    assert b_per_w % CH == 0
    n_ch = b_per_w // CH
    mesh = plsc.VectorSubcoreMesh(core_axis_name="c", subcore_axis_name="s")

    @functools.partial(
        pl.kernel,
        mesh=mesh,
        out_type=jax.ShapeDtypeStruct((B, D), jnp.float32),
        scratch_types=[
            pltpu.VMEM((CH,), jnp.int32),
            pltpu.VMEM((CH, D), jnp.float32),
            pltpu.SemaphoreType.DMA,
        ],
        compiler_params=pltpu.CompilerParams(use_tc_tiling_on_sc=False),
    )
    def k(table_hbm, idx_hbm, out_hbm, idx_v, rows_v, sem):
        wid = lax.axis_index("s") * NC + lax.axis_index("c")
        base = wid * b_per_w

        def body(i, carry):
            off = base + i * CH
            pltpu.sync_copy(idx_hbm.at[pl.ds(off, CH)], idx_v)
            pltpu.async_copy(table_hbm.at[idx_v], rows_v, sem).wait()
            pltpu.sync_copy(rows_v, out_hbm.at[pl.ds(off, CH)])
            return carry

        lax.fori_loop(0, n_ch, body, 0)

    return k


def kernel(x, embeddings):
    V, D = embeddings.shape
    B = x.shape[0] * x.shape[1]
    idx = x.reshape(-1).astype(jnp.int32)
    out = _make_gather(V, D, B, 1600)(embeddings, idx)
    return out.reshape(x.shape + (D,))


# speedup vs baseline: 1.1101x; 1.0063x over previous
import functools

import jax
import jax.numpy as jnp
from jax import lax
from jax.experimental import pallas as pl
from jax.experimental.pallas import tpu as pltpu
from jax.experimental.pallas import tpu_sc as plsc


@functools.lru_cache(maxsize=None)
def _make_gather(V, D, B, CH):
    info = plsc.get_sparse_core_info()
    NC, NS = info.num_cores, info.num_subcores
    NW = NC * NS
    assert B % NW == 0
    b_per_w = B // NW
    assert b_per_w % (2 * CH) == 0
    n_pair = b_per_w // (2 * CH)
    mesh = plsc.VectorSubcoreMesh(core_axis_name="c", subcore_axis_name="s")

    @functools.partial(
        pl.kernel,
        mesh=mesh,
        out_type=jax.ShapeDtypeStruct((B, D), jnp.float32),
        scratch_types=[
            pltpu.VMEM((CH,), jnp.int32),
            pltpu.VMEM((CH,), jnp.int32),
            pltpu.VMEM((CH, D), jnp.float32),
            pltpu.VMEM((CH, D), jnp.float32),
            pltpu.SemaphoreType.DMA,
            pltpu.SemaphoreType.DMA,
            pltpu.SemaphoreType.DMA,
            pltpu.SemaphoreType.DMA,
            pltpu.SemaphoreType.DMA,
        ],
        compiler_params=pltpu.CompilerParams(use_tc_tiling_on_sc=False),
    )
    def k(table_hbm, idx_hbm, out_hbm, idx0, idx1, rows0, rows1,
          si0, si1, sg, so0, so1):
        wid = lax.axis_index("s") * NC + lax.axis_index("c")
        base = wid * b_per_w
        idx_v = (idx0, idx1)
        rows_v = (rows0, rows1)
        sem_i = (si0, si1)
        sem_o = (so0, so1)

        pltpu.async_copy(idx_hbm.at[pl.ds(base, CH)], idx0, si0)

        @pl.loop(0, n_pair)
        def _(j):
            for b in range(2):
                i = 2 * j + b
                off = base + i * CH
                pltpu.make_async_copy(
                    idx_hbm.at[pl.ds(off, CH)], idx_v[b], sem_i[b]).wait()
                @pl.when(i >= 2)
                def _():
                    pltpu.make_async_copy(
                        rows_v[b], out_hbm.at[pl.ds(off - 2 * CH, CH)],
                        sem_o[b]).wait()
                gcopy = pltpu.make_async_copy(
                    table_hbm.at[idx_v[b]], rows_v[b], sg)
                gcopy.start()
                @pl.when(i + 1 < 2 * n_pair)
                def _():
                    pltpu.async_copy(
                        idx_hbm.at[pl.ds(off + CH, CH)],
                        idx_v[1 - b], sem_i[1 - b])
                gcopy.wait()
                pltpu.async_copy(
                    rows_v[b], out_hbm.at[pl.ds(off, CH)], sem_o[b])

        n_ch = 2 * n_pair
        pltpu.make_async_copy(
            rows0, out_hbm.at[pl.ds(base + (n_ch - 2) * CH, CH)], so0).wait()
        pltpu.make_async_copy(
            rows1, out_hbm.at[pl.ds(base + (n_ch - 1) * CH, CH)], so1).wait()

    return k


def kernel(x, embeddings):
    V, D = embeddings.shape
    B = x.shape[0] * x.shape[1]
    idx = x.reshape(-1).astype(jnp.int32)
    out = _make_gather(V, D, B, 1600)(embeddings, idx)
    return out.reshape(x.shape + (D,))


# speedup vs baseline: 1.6030x; 1.4440x over previous
import functools

import jax
import jax.numpy as jnp
from jax import lax
from jax.experimental import pallas as pl
from jax.experimental.pallas import tpu as pltpu
from jax.experimental.pallas import tpu_sc as plsc

_L = 16


@functools.lru_cache(maxsize=None)
def _make_lookup(V, D, B0, B1):
    info = plsc.get_sparse_core_info()
    NC, NS = info.num_cores, info.num_subcores
    NW = NC * NS
    assert V % 4 == 0 and D == 32
    assert B0 % NW == 0
    W = B0 // NW
    CB = 256
    assert W % CB == 0
    n_sub = W // CB
    mesh = plsc.VectorSubcoreMesh(core_axis_name="c", subcore_axis_name="s")

    @functools.partial(
        pl.kernel,
        mesh=mesh,
        out_type=jax.ShapeDtypeStruct((B1, D, B0), jnp.float32),
        scratch_types=[
            pltpu.VMEM((B1, W), jnp.int32),
            pltpu.VMEM((CB,), jnp.int32),
            pltpu.VMEM((CB,), jnp.int32),
            pltpu.VMEM((CB, 128), jnp.float32),
            pltpu.VMEM((CB, 128), jnp.float32),
            pltpu.VMEM((D, CB), jnp.float32),
            pltpu.VMEM((D, CB), jnp.float32),
            pltpu.SemaphoreType.DMA,
            pltpu.SemaphoreType.DMA,
            pltpu.SemaphoreType.DMA,
            pltpu.SemaphoreType.DMA,
        ],
        compiler_params=pltpu.CompilerParams(
            use_tc_tiling_on_sc=True, needs_layout_passes=False),
    )
    def k(tbl_hbm, xt_hbm, out_hbm, xb, gi0, gi1, rows0, rows1, ob0, ob1,
          sg0, sg1, so0, so1):
        wid = lax.axis_index("s") * NC + lax.axis_index("c")
        b0_base = wid * W
        gi = (gi0, gi1)
        rows = (rows0, rows1)
        ob = (ob0, ob1)
        sg = (sg0, sg1)
        so = (so0, so1)
        lane = lax.iota(jnp.int32, _L)
        n_ch = B1 * n_sub

        pltpu.sync_copy(xt_hbm.at[:, pl.ds(b0_base, W)], xb)

        def build_gidx(c, buf):
            b1 = c // n_sub
            sub = c % n_sub

            @pl.loop(0, CB // _L)
            def _(g):
                v = xb[b1, pl.ds(sub * CB + g * _L, _L)]
                gi[buf][pl.ds(g * _L, _L)] = lax.shift_right_logical(v, 2)

        def gather_copy(buf):
            return pltpu.make_async_copy(
                tbl_hbm.at[gi[buf]], rows[buf], sg[buf])

        def body(c, buf, nbuf):
            b1 = c // n_sub
            sub = c % n_sub
            @pl.when(c + 1 < n_ch)
            def _():
                build_gidx(c + 1, nbuf)
                gather_copy(nbuf).start()
            gather_copy(buf).wait()
            @pl.when(c >= 2)
            def _():
                pltpu.make_async_copy(
                    ob[buf], out_hbm.at[0, :, pl.ds(0, CB)], so[buf]).wait()
            @pl.loop(0, CB // _L)
            def _(g):
                v = xb[b1, pl.ds(sub * CB + g * _L, _L)]
                colbase = lax.rem(v, 4) * D
                rowid = g * _L + lane

                @pl.loop(0, D, unroll=8)
                def _(d):
                    vals = plsc.load_gather(rows[buf], [rowid, colbase + d])
                    ob[buf][d, pl.ds(g * _L, _L)] = vals

            pltpu.async_copy(
                ob[buf],
                out_hbm.at[b1, :, pl.ds(b0_base + sub * CB, CB)],
                so[buf])

        build_gidx(0, 0)
        gather_copy(0).start()

        @pl.loop(0, n_ch)
        def _(c):
            buf = lax.rem(c, 2)

            @pl.when(buf == 0)
            def _():
                body(c, 0, 1)

            @pl.when(buf == 1)
            def _():
                body(c, 1, 0)

        pltpu.make_async_copy(
            ob0, out_hbm.at[0, :, pl.ds(0, CB)], so0).wait()
        pltpu.make_async_copy(
            ob1, out_hbm.at[0, :, pl.ds(0, CB)], so1).wait()

    return k


def kernel(x, embeddings):
    V, D = embeddings.shape
    B0, B1 = x.shape
    tbl4 = embeddings.reshape(V // 4, 4 * D)
    xt = x.T.astype(jnp.int32)
    out = _make_lookup(V, D, B0, B1)(tbl4, xt)
    return out.transpose(2, 0, 1)


# speedup vs baseline: 1.6105x; 1.0047x over previous
import functools

import jax
import jax.numpy as jnp
from jax import lax
from jax.experimental import pallas as pl
from jax.experimental.pallas import tpu as pltpu
from jax.experimental.pallas import tpu_sc as plsc

_L = 16


@functools.lru_cache(maxsize=None)
def _make_lookup(V, D, B0, B1):
    info = plsc.get_sparse_core_info()
    NC, NS = info.num_cores, info.num_subcores
    NW = NC * NS
    assert V % 4 == 0 and D == 32
    assert B0 % NW == 0
    W = B0 // NW
    CB = 256
    assert W % CB == 0
    n_sub = W // CB
    mesh = plsc.VectorSubcoreMesh(core_axis_name="c", subcore_axis_name="s")

    @functools.partial(
        pl.kernel,
        mesh=mesh,
        out_type=jax.ShapeDtypeStruct((B1, D, B0), jnp.float32),
        scratch_types=[
            pltpu.VMEM((B1, W), jnp.int32),
            pltpu.VMEM((CB,), jnp.int32),
            pltpu.VMEM((CB,), jnp.int32),
            pltpu.VMEM((CB,), jnp.int32),
            pltpu.VMEM((CB,), jnp.int32),
            pltpu.VMEM((CB, 128), jnp.float32),
            pltpu.VMEM((CB, 128), jnp.float32),
            pltpu.VMEM((D, CB), jnp.float32),
            pltpu.VMEM((D, CB), jnp.float32),
            pltpu.SemaphoreType.DMA,
            pltpu.SemaphoreType.DMA,
            pltpu.SemaphoreType.DMA,
            pltpu.SemaphoreType.DMA,
        ],
        compiler_params=pltpu.CompilerParams(
            use_tc_tiling_on_sc=True, needs_layout_passes=False),
    )
    def k(tbl_hbm, xt_hbm, out_hbm, xb, gi0, gi1, cb0, cb1,
          rows0, rows1, ob0, ob1, sg0, sg1, so0, so1):
        wid = lax.axis_index("s") * NC + lax.axis_index("c")
        b0_base = wid * W
        gi = (gi0, gi1)
        cb = (cb0, cb1)
        rows = (rows0, rows1)
        ob = (ob0, ob1)
        sg = (sg0, sg1)
        so = (so0, so1)
        lane = lax.iota(jnp.int32, _L)
        n_ch = B1 * n_sub

        pltpu.sync_copy(xt_hbm.at[:, pl.ds(b0_base, W)], xb)

        def build_gidx(c, buf):
            b1 = c // n_sub
            sub = c % n_sub

            @pl.loop(0, CB // _L)
            def _(g):
                v = xb[b1, pl.ds(sub * CB + g * _L, _L)]
                gi[buf][pl.ds(g * _L, _L)] = lax.shift_right_logical(v, 2)
                cb[buf][pl.ds(g * _L, _L)] = lax.rem(v, 4) * D

        def gather_copy(buf):
            return pltpu.make_async_copy(
                tbl_hbm.at[gi[buf]], rows[buf], sg[buf])

        def body(c, buf, nbuf):
            b1 = c // n_sub
            sub = c % n_sub
            @pl.when(c + 1 < n_ch)
            def _():
                build_gidx(c + 1, nbuf)
                gather_copy(nbuf).start()
            gather_copy(buf).wait()
            @pl.when(c >= 2)
            def _():
                pltpu.make_async_copy(
                    ob[buf], out_hbm.at[0, :, pl.ds(0, CB)], so[buf]).wait()
            @pl.loop(0, CB // _L, unroll=2)
            def _(g):
                colbase = cb[buf][pl.ds(g * _L, _L)]
                rowid = g * _L + lane
                for d in range(D):
                    vals = plsc.load_gather(rows[buf], [rowid, colbase + d])
                    ob[buf][d, pl.ds(g * _L, _L)] = vals

            pltpu.async_copy(
                ob[buf],
                out_hbm.at[b1, :, pl.ds(b0_base + sub * CB, CB)],
                so[buf])

        build_gidx(0, 0)
        gather_copy(0).start()

        @pl.loop(0, n_ch)
        def _(c):
            buf = lax.rem(c, 2)

            @pl.when(buf == 0)
            def _():
                body(c, 0, 1)

            @pl.when(buf == 1)
            def _():
                body(c, 1, 0)

        pltpu.make_async_copy(
            ob0, out_hbm.at[0, :, pl.ds(0, CB)], so0).wait()
        pltpu.make_async_copy(
            ob1, out_hbm.at[0, :, pl.ds(0, CB)], so1).wait()

    return k


def kernel(x, embeddings):
    V, D = embeddings.shape
    B0, B1 = x.shape
    tbl4 = embeddings.reshape(V // 4, 4 * D)
    xt = x.T.astype(jnp.int32)
    out = _make_lookup(V, D, B0, B1)(tbl4, xt)
    return out.transpose(2, 0, 1)


# speedup vs baseline: 1.6436x; 1.0205x over previous
import functools

import jax
import jax.numpy as jnp
from jax import lax
from jax.experimental import pallas as pl
from jax.experimental.pallas import tpu as pltpu
from jax.experimental.pallas import tpu_sc as plsc

_L = 16


@functools.lru_cache(maxsize=None)
def _make_lookup(V, D, B0, B1):
    info = plsc.get_sparse_core_info()
    NC, NS = info.num_cores, info.num_subcores
    NW = NC * NS
    assert D == 32 and B0 % (NW * 128) == 0
    W = B0 // NW
    TC_W = W // 128
    mesh = plsc.VectorSubcoreMesh(core_axis_name="c", subcore_axis_name="s")

    @functools.partial(
        pl.kernel,
        mesh=mesh,
        out_type=jax.ShapeDtypeStruct((B1, D // 8, B0 // 128, 8, 128),
                                      jnp.float32),
        scratch_types=[
            pltpu.VMEM((B1, W), jnp.int32),
            pltpu.VMEM((W, D), jnp.float32),
            pltpu.VMEM((W, D), jnp.float32),
            pltpu.VMEM((D // 8, TC_W, 8, 128), jnp.float32),
            pltpu.VMEM((D // 8, TC_W, 8, 128), jnp.float32),
            pltpu.SemaphoreType.DMA,
            pltpu.SemaphoreType.DMA,
            pltpu.SemaphoreType.DMA,
            pltpu.SemaphoreType.DMA,
        ],
        compiler_params=pltpu.CompilerParams(
            use_tc_tiling_on_sc=False, needs_layout_passes=False),
    )
    def k(tbl_hbm, xt_hbm, out_hbm, xb, rows0, rows1, ob0, ob1,
          sg0, sg1, so0, so1):
        wid = lax.axis_index("s") * NC + lax.axis_index("c")
        b0_base = wid * W
        tc0 = wid * TC_W
        rows = (rows0, rows1)
        ob = (ob0, ob1)
        sg = (sg0, sg1)
        so = (so0, so1)
        lane = lax.iota(jnp.int32, _L)

        pltpu.sync_copy(xt_hbm.at[:, pl.ds(b0_base, W)], xb)

        def gather_copy(c, buf):
            return pltpu.make_async_copy(
                tbl_hbm.at[xb.at[c]], rows[buf], sg[buf])

        def body(c, buf):
            @pl.when(c + 1 < B1)
            def _():
                gather_copy(c + 1, 1 - buf).start()
            gather_copy(c, buf).wait()
            @pl.when(c >= 2)
            def _():
                pltpu.make_async_copy(
                    ob[buf], out_hbm.at[0, :, pl.ds(0, TC_W)],
                    so[buf]).wait()
            @pl.loop(0, W // _L, unroll=2)
            def _(g):
                rowid = g * _L + lane
                for d in range(D):
                    vals = plsc.load_gather(
                        rows[buf],
                        [rowid, jnp.full((_L,), d, jnp.int32)])
                    ob[buf][d // 8, g // 8, d % 8,
                            pl.ds((g % 8) * _L, _L)] = vals

            pltpu.async_copy(
                ob[buf], out_hbm.at[c, :, pl.ds(tc0, TC_W)], so[buf])

        gather_copy(0, 0).start()

        @pl.loop(0, B1)
        def _(c):
            buf = lax.rem(c, 2)

            @pl.when(buf == 0)
            def _():
                body(c, 0)

            @pl.when(buf == 1)
            def _():
                body(c, 1)

        pltpu.make_async_copy(
            ob0, out_hbm.at[0, :, pl.ds(0, TC_W)], so0).wait()
        pltpu.make_async_copy(
            ob1, out_hbm.at[0, :, pl.ds(0, TC_W)], so1).wait()

    return k


def kernel(x, embeddings):
    V, D = embeddings.shape
    B0, B1 = x.shape
    xt = x.T.astype(jnp.int32)
    out5 = _make_lookup(V, D, B0, B1)(embeddings, xt)
    out = out5.transpose(0, 1, 3, 2, 4).reshape(B1, D, B0)
    return out.transpose(2, 0, 1)


# speedup vs baseline: 1.6500x; 1.0039x over previous
import functools

import jax
import jax.numpy as jnp
from jax import lax
from jax.experimental import pallas as pl
from jax.experimental.pallas import tpu as pltpu
from jax.experimental.pallas import tpu_sc as plsc

_L = 16


@functools.lru_cache(maxsize=None)
def _make_lookup(V, D, B0, B1):
    info = plsc.get_sparse_core_info()
    NC, NS = info.num_cores, info.num_subcores
    NW = NC * NS
    assert D == 32 and B0 % (NW * 128) == 0
    W = B0 // NW
    TC_W = W // 128
    mesh = plsc.VectorSubcoreMesh(core_axis_name="c", subcore_axis_name="s")

    @functools.partial(
        pl.kernel,
        mesh=mesh,
        out_type=jax.ShapeDtypeStruct((B1, D // 8, B0 // 128, 8, 128),
                                      jnp.float32),
        scratch_types=[
            pltpu.VMEM((B1, W), jnp.int32),
            pltpu.VMEM((W, D), jnp.float32),
            pltpu.VMEM((W, D), jnp.float32),
            pltpu.VMEM((D // 8, TC_W, 8, 128), jnp.float32),
            pltpu.VMEM((D // 8, TC_W, 8, 128), jnp.float32),
            pltpu.SemaphoreType.DMA,
            pltpu.SemaphoreType.DMA,
            pltpu.SemaphoreType.DMA,
            pltpu.SemaphoreType.DMA,
        ],
        compiler_params=pltpu.CompilerParams(
            use_tc_tiling_on_sc=False, needs_layout_passes=False),
    )
    def k(tbl_hbm, xt_hbm, out_hbm, xb, rows0, rows1, ob0, ob1,
          sg0, sg1, so0, so1):
        wid = lax.axis_index("s") * NC + lax.axis_index("c")
        b0_base = wid * W
        tc0 = wid * TC_W
        rows = (rows0, rows1)
        ob = (ob0, ob1)
        sg = (sg0, sg1)
        so = (so0, so1)
        lane = lax.iota(jnp.int32, _L)

        pltpu.sync_copy(xt_hbm.at[:, pl.ds(b0_base, W)], xb)

        def gather_copy(c, buf):
            return pltpu.make_async_copy(
                tbl_hbm.at[xb.at[c]], rows[buf], sg[buf])

        def body(c, buf):
            @pl.when(c + 1 < B1)
            def _():
                gather_copy(c + 1, 1 - buf).start()
            gather_copy(c, buf).wait()
            @pl.when(c >= 2)
            def _():
                pltpu.make_async_copy(
                    ob[buf], out_hbm.at[0, :, pl.ds(0, TC_W)],
                    so[buf]).wait()
            rowids = [g * _L + lane for g in range(W // _L)]

            @pl.loop(0, D)
            def _(d):
                tr = d // 8
                s = lax.rem(d, 8)
                dvec = jnp.full((_L,), 0, jnp.int32) + d
                for g in range(W // _L):
                    vals = plsc.load_gather(rows[buf], [rowids[g], dvec])
                    ob[buf][tr, g // 8, s, pl.ds((g % 8) * _L, _L)] = vals

            pltpu.async_copy(
                ob[buf], out_hbm.at[c, :, pl.ds(tc0, TC_W)], so[buf])

        gather_copy(0, 0).start()

        @pl.loop(0, B1)
        def _(c):
            buf = lax.rem(c, 2)

            @pl.when(buf == 0)
            def _():
                body(c, 0)

            @pl.when(buf == 1)
            def _():
                body(c, 1)

        pltpu.make_async_copy(
            ob0, out_hbm.at[0, :, pl.ds(0, TC_W)], so0).wait()
        pltpu.make_async_copy(
            ob1, out_hbm.at[0, :, pl.ds(0, TC_W)], so1).wait()

    return k


def kernel(x, embeddings):
    V, D = embeddings.shape
    B0, B1 = x.shape
    xt = x.T.astype(jnp.int32)
    out5 = _make_lookup(V, D, B0, B1)(embeddings, xt)
    out = out5.transpose(0, 1, 3, 2, 4).reshape(B1, D, B0)
    return out.transpose(2, 0, 1)


# speedup vs baseline: 2.5560x; 1.5491x over previous
import functools

import jax
import jax.numpy as jnp
from jax import lax
from jax.experimental import pallas as pl
from jax.experimental.pallas import tpu as pltpu
from jax.experimental.pallas import tpu_sc as plsc

_L = 16


@functools.lru_cache(maxsize=None)
def _make_lookup(V, D, B0, B1):
    info = plsc.get_sparse_core_info()
    NC, NS = info.num_cores, info.num_subcores
    NW = NC * NS
    assert D == 32 and B0 % (NW * 128) == 0
    W = B0 // NW
    TC_W = W // 128
    mesh = plsc.VectorSubcoreMesh(core_axis_name="c", subcore_axis_name="s")

    @functools.partial(
        pl.kernel,
        mesh=mesh,
        out_type=jax.ShapeDtypeStruct((B1, D // 8, B0 // 128, 8, 128),
                                      jnp.float32),
        scratch_types=[
            pltpu.VMEM((B1, W), jnp.int32),
            pltpu.VMEM((W, D), jnp.float32),
            pltpu.VMEM((W, D), jnp.float32),
            pltpu.VMEM((D // 8, TC_W, 8, 129), jnp.float32),
            pltpu.VMEM((D // 8, TC_W, 8, 129), jnp.float32),
            pltpu.SemaphoreType.DMA,
            pltpu.SemaphoreType.DMA,
            pltpu.SemaphoreType.DMA,
            pltpu.SemaphoreType.DMA,
        ],
        compiler_params=pltpu.CompilerParams(
            use_tc_tiling_on_sc=False, needs_layout_passes=False),
    )
    def k(tbl_hbm, xt_hbm, out_hbm, xb, rows0, rows1, ob0, ob1,
          sg0, sg1, so0, so1):
        wid = lax.axis_index("s") * NC + lax.axis_index("c")
        b0_base = wid * W
        tc0 = wid * TC_W
        rows = (rows0, rows1)
        ob = (ob0, ob1)
        sg = (sg0, sg1)
        so = (so0, so1)
        lane = lax.iota(jnp.int32, _L)

        pltpu.sync_copy(xt_hbm.at[:, pl.ds(b0_base, W)], xb)

        def gather_copy(c, buf):
            return pltpu.make_async_copy(
                tbl_hbm.at[xb.at[c]], rows[buf], sg[buf])

        def body(c, buf):
            @pl.when(c + 1 < B1)
            def _():
                gather_copy(c + 1, 1 - buf).start()
            gather_copy(c, buf).wait()
            @pl.when(c >= 2)
            def _():
                pltpu.make_async_copy(
                    ob[buf].at[:, :, :, pl.ds(0, 128)],
                    out_hbm.at[0, :, pl.ds(0, TC_W)],
                    so[buf]).wait()
            trv = (lane // 8, 2 + lane // 8)
            sv = lax.rem(lane, 8)

            @pl.loop(0, W, unroll=4)
            def _(j):
                jcv = jnp.full((_L,), 0, jnp.int32) + (j // 128)
                lv = jnp.full((_L,), 0, jnp.int32) + lax.rem(j, 128)
                for kk in range(2):
                    vals = rows[buf][j, pl.ds(kk * _L, _L)]
                    plsc.store_scatter(
                        ob[buf], [trv[kk], jcv, sv, lv], vals)

            pltpu.async_copy(
                ob[buf].at[:, :, :, pl.ds(0, 128)],
                out_hbm.at[c, :, pl.ds(tc0, TC_W)], so[buf])

        gather_copy(0, 0).start()

        @pl.loop(0, B1)
        def _(c):
            buf = lax.rem(c, 2)

            @pl.when(buf == 0)
            def _():
                body(c, 0)

            @pl.when(buf == 1)
            def _():
                body(c, 1)

        pltpu.make_async_copy(
            ob0.at[:, :, :, pl.ds(0, 128)],
            out_hbm.at[0, :, pl.ds(0, TC_W)], so0).wait()
        pltpu.make_async_copy(
            ob1.at[:, :, :, pl.ds(0, 128)],
            out_hbm.at[0, :, pl.ds(0, TC_W)], so1).wait()

    return k


def kernel(x, embeddings):
    V, D = embeddings.shape
    B0, B1 = x.shape
    xt = x.T.astype(jnp.int32)
    out5 = _make_lookup(V, D, B0, B1)(embeddings, xt)
    out = out5.transpose(0, 1, 3, 2, 4).reshape(B1, D, B0)
    return out.transpose(2, 0, 1)


# speedup vs baseline: 2.5750x; 1.0074x over previous
import functools

import jax
import jax.numpy as jnp
from jax import lax
from jax.experimental import pallas as pl
from jax.experimental.pallas import tpu as pltpu
from jax.experimental.pallas import tpu_sc as plsc

_L = 16


@functools.lru_cache(maxsize=None)
def _make_lookup(V, D, B0, B1):
    info = plsc.get_sparse_core_info()
    NC, NS = info.num_cores, info.num_subcores
    NW = NC * NS
    assert D == 32 and B0 % (NW * 128) == 0
    W = B0 // NW
    TC_W = W // 128
    mesh = plsc.VectorSubcoreMesh(core_axis_name="c", subcore_axis_name="s")

    @functools.partial(
        pl.kernel,
        mesh=mesh,
        out_type=jax.ShapeDtypeStruct((B1, D // 8, B0 // 128, 8, 128),
                                      jnp.float32),
        scratch_types=[
            pltpu.VMEM((B1, W), jnp.int32),
            pltpu.VMEM((W, D), jnp.float32),
            pltpu.VMEM((W, D), jnp.float32),
            pltpu.VMEM((D // 8, TC_W, 8, 129), jnp.float32),
            pltpu.VMEM((D // 8, TC_W, 8, 129), jnp.float32),
            pltpu.SemaphoreType.DMA,
            pltpu.SemaphoreType.DMA,
            pltpu.SemaphoreType.DMA,
            pltpu.SemaphoreType.DMA,
        ],
        compiler_params=pltpu.CompilerParams(
            use_tc_tiling_on_sc=False, needs_layout_passes=False),
    )
    def k(tbl_hbm, xt_hbm, out_hbm, xb, rows0, rows1, ob0, ob1,
          sg0, sg1, so0, so1):
        wid = lax.axis_index("s") * NC + lax.axis_index("c")
        b0_base = wid * W
        tc0 = wid * TC_W
        rows = (rows0, rows1)
        ob = (ob0, ob1)
        sg = (sg0, sg1)
        so = (so0, so1)
        lane = lax.iota(jnp.int32, _L)

        pltpu.sync_copy(xt_hbm.at[:, pl.ds(b0_base, W)], xb)

        def gather_copy(c, buf):
            return pltpu.make_async_copy(
                tbl_hbm.at[xb.at[c]], rows[buf], sg[buf])

        def body(c, buf):
            @pl.when(c + 1 < B1)
            def _():
                gather_copy(c + 1, 1 - buf).start()
            gather_copy(c, buf).wait()
            @pl.when(c >= 2)
            def _():
                pltpu.make_async_copy(
                    ob[buf].at[:, :, :, pl.ds(0, 128)],
                    out_hbm.at[0, :, pl.ds(0, TC_W)],
                    so[buf]).wait()
            trv = (lane // 8, 2 + lane // 8)
            sv = lax.rem(lane, 8)

            @pl.loop(0, W, unroll=8)
            def _(j):
                jcv = jnp.full((_L,), 0, jnp.int32) + (j // 128)
                lv = jnp.full((_L,), 0, jnp.int32) + lax.rem(j, 128)
                for kk in range(2):
                    vals = rows[buf][j, pl.ds(kk * _L, _L)]
                    plsc.store_scatter(
                        ob[buf], [trv[kk], jcv, sv, lv], vals)

            pltpu.async_copy(
                ob[buf].at[:, :, :, pl.ds(0, 128)],
                out_hbm.at[c, :, pl.ds(tc0, TC_W)], so[buf])

        gather_copy(0, 0).start()

        @pl.loop(0, B1)
        def _(c):
            buf = lax.rem(c, 2)

            @pl.when(buf == 0)
            def _():
                body(c, 0)

            @pl.when(buf == 1)
            def _():
                body(c, 1)

        pltpu.make_async_copy(
            ob0.at[:, :, :, pl.ds(0, 128)],
            out_hbm.at[0, :, pl.ds(0, TC_W)], so0).wait()
        pltpu.make_async_copy(
            ob1.at[:, :, :, pl.ds(0, 128)],
            out_hbm.at[0, :, pl.ds(0, TC_W)], so1).wait()

    return k


def kernel(x, embeddings):
    V, D = embeddings.shape
    B0, B1 = x.shape
    xt = x.T.astype(jnp.int32)
    out5 = _make_lookup(V, D, B0, B1)(embeddings, xt)
    out = out5.transpose(0, 1, 3, 2, 4).reshape(B1, D, B0)
    return out.transpose(2, 0, 1)


# speedup vs baseline: 2.6138x; 1.0151x over previous
import functools

import jax
import jax.numpy as jnp
from jax import lax
from jax.experimental import pallas as pl
from jax.experimental.pallas import tpu as pltpu
from jax.experimental.pallas import tpu_sc as plsc

_L = 16


@functools.lru_cache(maxsize=None)
def _make_lookup(V, D, B0, B1):
    info = plsc.get_sparse_core_info()
    NC, NS = info.num_cores, info.num_subcores
    NW = NC * NS
    assert D == 32 and B0 % (NW * 128) == 0
    W = B0 // NW
    TC_W = W // 128
    mesh = plsc.VectorSubcoreMesh(core_axis_name="c", subcore_axis_name="s")

    @functools.partial(
        pl.kernel,
        mesh=mesh,
        out_type=jax.ShapeDtypeStruct((B1, D // 8, B0 // 128, 8, 128),
                                      jnp.float32),
        scratch_types=[
            pltpu.VMEM((B1, W), jnp.int32),
            pltpu.VMEM((W, D), jnp.float32),
            pltpu.VMEM((W, D), jnp.float32),
            pltpu.VMEM((D // 8, TC_W, 10, 129), jnp.float32),
            pltpu.VMEM((D // 8, TC_W, 10, 129), jnp.float32),
            pltpu.SemaphoreType.DMA,
            pltpu.SemaphoreType.DMA,
            pltpu.SemaphoreType.DMA,
            pltpu.SemaphoreType.DMA,
        ],
        compiler_params=pltpu.CompilerParams(
            use_tc_tiling_on_sc=False, needs_layout_passes=False),
    )
    def k(tbl_hbm, xt_hbm, out_hbm, xb, rows0, rows1, ob0, ob1,
          sg0, sg1, so0, so1):
        wid = lax.axis_index("s") * NC + lax.axis_index("c")
        b0_base = wid * W
        tc0 = wid * TC_W
        rows = (rows0, rows1)
        ob = (ob0, ob1)
        sg = (sg0, sg1)
        so = (so0, so1)
        lane = lax.iota(jnp.int32, _L)

        pltpu.sync_copy(xt_hbm.at[:, pl.ds(b0_base, W)], xb)

        def gather_copy(c, buf):
            return pltpu.make_async_copy(
                tbl_hbm.at[xb.at[c]], rows[buf], sg[buf])

        def body(c, buf):
            @pl.when(c + 1 < B1)
            def _():
                gather_copy(c + 1, 1 - buf).start()
            gather_copy(c, buf).wait()
            @pl.when(c >= 2)
            def _():
                pltpu.make_async_copy(
                    ob[buf].at[:, :, pl.ds(0, 8), pl.ds(0, 128)],
                    out_hbm.at[0, :, pl.ds(0, TC_W)],
                    so[buf]).wait()
            trv = (lane // 8, 2 + lane // 8)
            sv = lax.rem(lane, 8)

            @pl.loop(0, W, unroll=8)
            def _(j):
                jcv = jnp.full((_L,), 0, jnp.int32) + (j // 128)
                lv = jnp.full((_L,), 0, jnp.int32) + lax.rem(j, 128)
                for kk in range(2):
                    vals = rows[buf][j, pl.ds(kk * _L, _L)]
                    plsc.store_scatter(
                        ob[buf], [trv[kk], jcv, sv, lv], vals)

            pltpu.async_copy(
                ob[buf].at[:, :, pl.ds(0, 8), pl.ds(0, 128)],
                out_hbm.at[c, :, pl.ds(tc0, TC_W)], so[buf])

        gather_copy(0, 0).start()

        @pl.loop(0, B1)
        def _(c):
            buf = lax.rem(c, 2)

            @pl.when(buf == 0)
            def _():
                body(c, 0)

            @pl.when(buf == 1)
            def _():
                body(c, 1)

        pltpu.make_async_copy(
            ob0.at[:, :, pl.ds(0, 8), pl.ds(0, 128)],
            out_hbm.at[0, :, pl.ds(0, TC_W)], so0).wait()
        pltpu.make_async_copy(
            ob1.at[:, :, pl.ds(0, 8), pl.ds(0, 128)],
            out_hbm.at[0, :, pl.ds(0, TC_W)], so1).wait()

    return k


def kernel(x, embeddings):
    V, D = embeddings.shape
    B0, B1 = x.shape
    xt = x.T.astype(jnp.int32)
    out5 = _make_lookup(V, D, B0, B1)(embeddings, xt)
    out = out5.transpose(0, 1, 3, 2, 4).reshape(B1, D, B0)
    return out.transpose(2, 0, 1)
